# baseline (speedup 1.0000x reference)
import jax
import jax.numpy as jnp
from jax.experimental import pallas as pl

_EPS = 1e-5


def _mm_body(x_ref, w_ref, o_ref):
    o_ref[...] = jnp.dot(
        x_ref[...], w_ref[...], preferred_element_type=jnp.float32
    ).astype(o_ref.dtype)


def _agg_quant_body(adj_ref, p_ref, b_ref, h_ref, q_ref):
    ab = adj_ref[...].astype(jnp.bfloat16)
    q_ref[...] = ab
    h_ref[...] = (
        jnp.dot(ab, p_ref[...], preferred_element_type=jnp.float32) + b_ref[...]
    )


def _agg_body(adj_ref, p_ref, b_ref, h_ref):
    h_ref[...] = (
        jnp.dot(adj_ref[...], p_ref[...], preferred_element_type=jnp.float32)
        + b_ref[...]
    )


def _agg_lsm_body(adj_ref, p_ref, b_ref, o_ref):
    h = (
        jnp.dot(adj_ref[...], p_ref[...], preferred_element_type=jnp.float32)
        + b_ref[...]
    )
    mx = jnp.max(h, axis=1, keepdims=True)
    lse = jnp.log(jnp.sum(jnp.exp(h - mx), axis=1, keepdims=True))
    o_ref[...] = h - mx - lse


def _bn_relu_mm_body(h_ref, g_ref, be_ref, w_ref, o_ref):
    h = h_ref[...]
    m = jnp.mean(h, axis=0, keepdims=True)
    c = h - m
    v = jnp.mean(c * c, axis=0, keepdims=True)
    hn = jnp.maximum(c * jax.lax.rsqrt(v + _EPS) * g_ref[...] + be_ref[...], 0.0)
    o_ref[...] = jnp.dot(
        hn, w_ref[...], preferred_element_type=jnp.float32
    ).astype(o_ref.dtype)


def _aggregate(body, adj, p, b, rb, out_shapes, extra_out=False):
    n = adj.shape[0]
    hd = p.shape[1]
    in_specs = [
        pl.BlockSpec((rb, n), lambda i: (i, 0)),
        pl.BlockSpec((n, hd), lambda i: (0, 0)),
        pl.BlockSpec((1, hd), lambda i: (0, 0)),
    ]
    out_specs = [pl.BlockSpec((rb, hd), lambda i: (i, 0))]
    if extra_out:
        out_specs.append(pl.BlockSpec((rb, n), lambda i: (i, 0)))
    if not extra_out:
        out_specs = out_specs[0]
        out_shapes = out_shapes[0]
    return pl.pallas_call(
        body,
        grid=(n // rb,),
        in_specs=in_specs,
        out_specs=out_specs,
        out_shape=out_shapes,
    )(adj, p, b.reshape(1, -1))


def kernel(x, adj, W1, b1, g1, be1, W2, b2, g2, be2, W3, b3):
    n, _ = x.shape
    hdim = W1.shape[1]
    cdim = W3.shape[1]
    f32 = jnp.float32
    bf16 = jnp.bfloat16
    rb = 200 if n % 200 == 0 else n

    p1 = pl.pallas_call(
        _mm_body, out_shape=jax.ShapeDtypeStruct((n, hdim), bf16)
    )(x, W1)

    h1, qadj = _aggregate(
        _agg_quant_body, adj, p1, b1, rb,
        [jax.ShapeDtypeStruct((n, hdim), f32),
         jax.ShapeDtypeStruct((n, n), bf16)],
        extra_out=True,
    )

    p2 = pl.pallas_call(
        _bn_relu_mm_body, out_shape=jax.ShapeDtypeStruct((n, hdim), bf16)
    )(h1, g1.reshape(1, -1), be1.reshape(1, -1), W2)

    h2 = _aggregate(
        _agg_body, qadj, p2, b2, rb,
        [jax.ShapeDtypeStruct((n, hdim), f32)],
    )

    p3 = pl.pallas_call(
        _bn_relu_mm_body, out_shape=jax.ShapeDtypeStruct((n, cdim), bf16)
    )(h2, g2.reshape(1, -1), be2.reshape(1, -1), W3)

    out = _aggregate(
        _agg_lsm_body, qadj, p3, b3, rb,
        [jax.ShapeDtypeStruct((n, cdim), f32)],
    )
    return out
